# Initial kernel scaffold; baseline (speedup 1.0000x reference)
import jax
import jax.numpy as jnp
from jax.experimental import pallas as pl


def kernel(anchors, pred_bbox_deltas, objectness):
    raise NotImplementedError("write your pallas kernel here")


# speedup vs baseline: 51.2268x; 51.2268x over previous
import functools
import math

import jax
import jax.numpy as jnp
from jax.experimental import pallas as pl
from jax.experimental.pallas import tpu as pltpu

_IMG_H = 800.0
_IMG_W = 800.0
_PRE_NMS = 2000
_POST_NMS = 1000
_NMS_THRESH = 0.7
_N = 20000
_NPAD = 20480
_CPAD = 2048
_OPAD = 1024
_BBOX_XFORM_CLIP = math.log(1000.0 / 16)


def _decode_kernel(anch_ref, dobj_ref, out_ref):
    ax1 = anch_ref[0:1, :]
    ay1 = anch_ref[1:2, :]
    ax2 = anch_ref[2:3, :]
    ay2 = anch_ref[3:4, :]
    dx = dobj_ref[0:1, :]
    dy = dobj_ref[1:2, :]
    dw = dobj_ref[2:3, :]
    dh = dobj_ref[3:4, :]
    obj = dobj_ref[4:5, :]

    widths = ax2 - ax1
    heights = ay2 - ay1
    ctr_x = ax1 + 0.5 * widths
    ctr_y = ay1 + 0.5 * heights
    dw = jnp.minimum(dw, _BBOX_XFORM_CLIP)
    dh = jnp.minimum(dh, _BBOX_XFORM_CLIP)
    pred_ctr_x = dx * widths + ctr_x
    pred_ctr_y = dy * heights + ctr_y
    pred_w = jnp.exp(dw) * widths
    pred_h = jnp.exp(dh) * heights

    x1 = jnp.clip(pred_ctr_x - 0.5 * pred_w, 0.0, _IMG_W)
    y1 = jnp.clip(pred_ctr_y - 0.5 * pred_h, 0.0, _IMG_H)
    x2 = jnp.clip(pred_ctr_x + 0.5 * pred_w, 0.0, _IMG_W)
    y2 = jnp.clip(pred_ctr_y + 0.5 * pred_h, 0.0, _IMG_H)

    lane = jax.lax.broadcasted_iota(jnp.int32, x1.shape, 1)
    valid = ((x2 - x1) * (y2 - y1) > 1.0) & (lane < _N)
    score = jnp.where(valid, obj, -jnp.inf)

    out_ref[0:1, :] = x1
    out_ref[1:2, :] = y1
    out_ref[2:3, :] = x2
    out_ref[3:4, :] = y2
    out_ref[4:5, :] = score
    out_ref[5:8, :] = jnp.zeros_like(out_ref[5:8, :])


def _nms_kernel(cand_ref, candt_ref, out_ref, a_ref):
    x1r = cand_ref[0:1, :]
    y1r = cand_ref[1:2, :]
    x2r = cand_ref[2:3, :]
    y2r = cand_ref[3:4, :]
    x1c = candt_ref[:, 0:1]
    y1c = candt_ref[:, 1:2]
    x2c = candt_ref[:, 2:3]
    y2c = candt_ref[:, 3:4]
    sc_c = candt_ref[:, 4:5]

    area_r = (x2r - x1r) * (y2r - y1r)
    area_c = (x2c - x1c) * (y2c - y1c)

    col = jax.lax.broadcasted_iota(jnp.int32, (1, _CPAD), 1)
    row = jax.lax.broadcasted_iota(jnp.int32, (_CPAD, 1), 0)

    blk = 256
    for b in range(_CPAD // blk):
        r0 = b * blk
        bx1 = x1c[r0:r0 + blk, :]
        by1 = y1c[r0:r0 + blk, :]
        bx2 = x2c[r0:r0 + blk, :]
        by2 = y2c[r0:r0 + blk, :]
        barea = area_c[r0:r0 + blk, :]
        ltx = jnp.maximum(bx1, x1r)
        lty = jnp.maximum(by1, y1r)
        rbx = jnp.minimum(bx2, x2r)
        rby = jnp.minimum(by2, y2r)
        wx = jnp.clip(rbx - ltx, 0.0, None)
        wy = jnp.clip(rby - lty, 0.0, None)
        inter = wx * wy
        union = barea + area_r - inter
        iou = inter / jnp.maximum(union, 1e-9)
        brow = row[r0:r0 + blk, :]
        mask = (iou > _NMS_THRESH) & (col < brow) & (col < _PRE_NMS) & (brow < _PRE_NMS)
        a_ref[r0:r0 + blk, :] = mask.astype(jnp.float32)

    valid_row = (row < _PRE_NMS).astype(jnp.float32)

    def cond(carry):
        _, changed, it = carry
        return changed & (it < _CPAD + 2)

    def body(carry):
        k, _, it = carry
        s = jax.lax.dot_general(
            a_ref[...], k, (((1,), (0,)), ((), ())),
            preferred_element_type=jnp.float32)
        k_new = jnp.where(s > 0.0, 0.0, valid_row)
        changed = jnp.any(k_new != k)
        return k_new, changed, it + 1

    k0 = valid_row
    keep, _, _ = jax.lax.while_loop(cond, body, (k0, jnp.bool_(True), jnp.int32(0)))

    m = (keep > 0.0) & (sc_c > -jnp.inf) & (row < _PRE_NMS)
    nm = (~m) & (row < _PRE_NMS)
    mf = m.astype(jnp.float32)
    nmf = nm.astype(jnp.float32)

    for b in range(_CPAD // blk):
        r0 = b * blk
        brow = row[r0:r0 + blk, :]
        a_ref[r0:r0 + blk, :] = ((col < brow) & (col < _PRE_NMS)).astype(jnp.float32)

    lt = a_ref[...]
    cnt_m = jax.lax.dot_general(lt, mf, (((1,), (0,)), ((), ())),
                                preferred_element_type=jnp.float32)
    cnt_nm = jax.lax.dot_general(lt, nmf, (((1,), (0,)), ((), ())),
                                 preferred_element_type=jnp.float32)
    n_m = jnp.sum(mf)
    rank = jnp.where(m, cnt_m, n_m + cnt_nm)
    rank = jnp.where(row < _PRE_NMS, rank, 2.0 * _CPAD)

    out_col = jax.lax.broadcasted_iota(jnp.int32, (1, _OPAD), 1).astype(jnp.float32)
    sel = (rank == out_col) & (out_col < _POST_NMS)

    neg = -jnp.inf
    score_val = jnp.where(m, sc_c, neg)
    out_ref[0:1, :] = jnp.max(jnp.where(sel, x1c, neg), axis=0, keepdims=True)
    out_ref[1:2, :] = jnp.max(jnp.where(sel, y1c, neg), axis=0, keepdims=True)
    out_ref[2:3, :] = jnp.max(jnp.where(sel, x2c, neg), axis=0, keepdims=True)
    out_ref[3:4, :] = jnp.max(jnp.where(sel, y2c, neg), axis=0, keepdims=True)
    out_ref[4:5, :] = jnp.max(jnp.where(sel, score_val, neg), axis=0, keepdims=True)
    out_ref[5:8, :] = jnp.zeros_like(out_ref[5:8, :])


@functools.partial(jax.jit, static_argnames=("interpret",))
def _run(anchors, pred_bbox_deltas, objectness, interpret=False):
    f32 = jnp.float32
    anch = jnp.zeros((8, _NPAD), f32).at[0:4, :_N].set(anchors.T.astype(f32))
    dobj = (jnp.zeros((8, _NPAD), f32)
            .at[0:4, :_N].set(pred_bbox_deltas.T.astype(f32))
            .at[4, :_N].set(objectness.astype(f32)))

    dec = pl.pallas_call(
        _decode_kernel,
        out_shape=jax.ShapeDtypeStruct((8, _NPAD), f32),
        interpret=interpret,
    )(anch, dobj)

    scores = dec[4, :_N]
    top_scores, top_idx = jax.lax.top_k(scores, _PRE_NMS)
    top_cols = dec[0:4, :_N][:, top_idx]

    cand = jnp.full((8, _CPAD), 0.0, f32)
    cand = cand.at[0:4, :_PRE_NMS].set(top_cols)
    cand = cand.at[4, :_PRE_NMS].set(top_scores)
    cand = cand.at[4, _PRE_NMS:].set(-jnp.inf)
    candt = cand.T

    out = pl.pallas_call(
        _nms_kernel,
        out_shape=jax.ShapeDtypeStruct((8, _OPAD), f32),
        scratch_shapes=[pltpu.VMEM((_CPAD, _CPAD), f32)],
        interpret=interpret,
    )(cand, candt)

    return out[0:5, :_POST_NMS].T


def kernel(anchors, pred_bbox_deltas, objectness):
    return _run(anchors, pred_bbox_deltas, objectness)
